# baseline (speedup 1.0000x reference)
import functools

import jax
import jax.numpy as jnp
from jax import lax
from jax.experimental import pallas as pl
from jax.experimental.pallas import tpu as pltpu
from jax.experimental.pallas import tpu_sc as plsc

_NUM_CORES = 2
_NUM_SUBCORES = 16
_NUM_WORKERS = _NUM_CORES * _NUM_SUBCORES


@functools.cache
def _make_sc_pool(B, L, D):
    D2 = 2 * D
    BPW = B // _NUM_WORKERS
    G = 4
    NG = BPW // G
    IDXG = G * L
    chunks = []
    off = 0
    while off < IDXG:
        ln = min(128, IDXG - off)
        chunks.append((off, ln))
        off += ln

    mesh = plsc.VectorSubcoreMesh(core_axis_name="c", subcore_axis_name="s")

    @functools.partial(
        pl.kernel,
        mesh=mesh,
        out_type=jax.ShapeDtypeStruct((B * D,), jnp.float32),
        scratch_types=[
            pltpu.VMEM((IDXG,), jnp.int32),
            pltpu.VMEM((IDXG,), jnp.int32),
            pltpu.VMEM((IDXG, D2), jnp.float32),
            pltpu.VMEM((IDXG, D2), jnp.float32),
            pltpu.VMEM((G * D,), jnp.float32),
            pltpu.SemaphoreType.DMA,
            pltpu.SemaphoreType.DMA,
        ],
    )
    def sc_pool(idx_ref, emb2_ref, out_ref,
                idx0, idx1, rows0, rows1, pooled,
                sem0, sem1):
        wid = lax.axis_index("s") * _NUM_CORES + lax.axis_index("c")
        row_base = wid * BPW
        idx_base = row_base * L
        idxs = (idx0, idx1)
        rows = (rows0, rows1)
        sems = (sem0, sem1)

        def issue(g, buf):
            off = pl.multiple_of(idx_base + g * IDXG, 8)
            pltpu.sync_copy(idx_ref.at[pl.ds(off, IDXG)], idxs[buf])
            for (o, ln) in chunks:
                pltpu.async_copy(
                    emb2_ref.at[idxs[buf].at[pl.ds(o, ln)]],
                    rows[buf].at[pl.ds(o, ln)],
                    sems[buf])

        def wait(buf):
            for (o, ln) in chunks:
                pltpu.make_async_copy(
                    emb2_ref.at[idxs[buf].at[pl.ds(o, ln)]],
                    rows[buf].at[pl.ds(o, ln)],
                    sems[buf]).wait()

        def reduce_store(g, buf):
            r = rows[buf]

            def body(c, carry):
                accs = None
                for l in range(L):
                    s = c * L + l
                    cur = [r[s, pl.ds(j * 16, 16)] for j in range(D // 16)]
                    if accs is None:
                        accs = cur
                    else:
                        accs = [a + b for a, b in zip(accs, cur)]
                for j in range(D // 16):
                    pooled[pl.ds(pl.multiple_of(c * D + j * 16, 16), 16)] = (
                        accs[j])
                return carry

            lax.fori_loop(0, G, body, 0)
            out_off = pl.multiple_of((row_base + g * G) * D, 8)
            pltpu.sync_copy(pooled, out_ref.at[pl.ds(out_off, G * D)])

        issue(0, 0)

        def pair_body(p, carry):
            g0 = 2 * p
            issue(g0 + 1, 1)
            wait(0)
            reduce_store(g0, 0)

            @pl.when(g0 + 2 < NG)
            def _():
                issue(g0 + 2, 0)

            wait(1)
            reduce_store(g0 + 1, 1)
            return carry

        lax.fori_loop(0, NG // 2, pair_body, 0)

    return sc_pool


@functools.cache
def _make_tc_transpose(V, D):
    VB = 2048

    def body(x_ref, e_ref, o_ref):
        o_ref[...] = jax.lax.dot_general(
            x_ref[...], e_ref[...],
            dimension_numbers=(((0,), (0,)), ((), ())),
            preferred_element_type=jnp.float32)

    return pl.pallas_call(
        body,
        grid=((V + VB - 1) // VB,),
        in_specs=[
            pl.BlockSpec((D, VB), lambda i: (0, i)),
            pl.BlockSpec((D, 2 * D), lambda i: (0, 0)),
        ],
        out_specs=pl.BlockSpec((VB, 2 * D), lambda i: (i, 0)),
        out_shape=jax.ShapeDtypeStruct((V, 2 * D), jnp.float32),
    )


@functools.cache
def _make_mlp(B, D, H1, H2):
    BT = 512

    def body(x_ref, w1_ref, b1_ref, w2_ref, b2_ref, w3_ref, b3_ref, o_ref):
        x = x_ref[...]
        h1 = jnp.maximum(
            jnp.dot(x, w1_ref[...], preferred_element_type=jnp.float32)
            + b1_ref[...], 0.0)
        h2 = jnp.maximum(
            jnp.dot(h1, w2_ref[...], preferred_element_type=jnp.float32)
            + b2_ref[...], 0.0)
        o_ref[...] = (
            jnp.dot(h2, w3_ref[...], preferred_element_type=jnp.float32)
            + b3_ref[...])

    return pl.pallas_call(
        body,
        grid=(B // BT,),
        in_specs=[
            pl.BlockSpec((BT, D), lambda i: (i, 0)),
            pl.BlockSpec((D, H1), lambda i: (0, 0)),
            pl.BlockSpec((1, H1), lambda i: (0, 0)),
            pl.BlockSpec((H1, H2), lambda i: (0, 0)),
            pl.BlockSpec((1, H2), lambda i: (0, 0)),
            pl.BlockSpec((H2, 128), lambda i: (0, 0)),
            pl.BlockSpec((1, 128), lambda i: (0, 0)),
        ],
        out_specs=pl.BlockSpec((BT, 128), lambda i: (i, 0)),
        out_shape=jax.ShapeDtypeStruct((B, 128), jnp.float32),
    )


def kernel(sentence, emb, W1, b1, W2, b2, W3, b3):
    B, L = sentence.shape
    V, D = emb.shape
    H1 = W1.shape[0]
    H2 = W2.shape[0]
    NOUT = W3.shape[0]

    idx = sentence.astype(jnp.int32).reshape(-1)
    eye_pad = jnp.concatenate(
        [jnp.eye(D, dtype=jnp.float32), jnp.zeros((D, D), jnp.float32)],
        axis=1)
    emb2 = _make_tc_transpose(V, D)(emb.astype(jnp.float32).T, eye_pad)
    pooled = _make_sc_pool(B, L, D)(idx, emb2).reshape(B, D)

    w3t = jnp.zeros((H2, 128), jnp.float32).at[:, :NOUT].set(W3.T)
    b3p = jnp.zeros((1, 128), jnp.float32).at[0, :NOUT].set(b3)
    out = _make_mlp(B, D, H1, H2)(
        pooled, W1.T, b1.reshape(1, H1), W2.T, b2.reshape(1, H2), w3t, b3p)
    return out[:, :NOUT]


# speedup vs baseline: 1.3488x; 1.3488x over previous
import functools

import jax
import jax.numpy as jnp
from jax import lax
from jax.experimental import pallas as pl
from jax.experimental.pallas import tpu as pltpu
from jax.experimental.pallas import tpu_sc as plsc

_NUM_CORES = 2
_NUM_SUBCORES = 16
_NUM_WORKERS = _NUM_CORES * _NUM_SUBCORES


@functools.cache
def _make_sc_pool(B, L, D):
    D2 = 2 * D
    BPW = B // _NUM_WORKERS
    G = 4
    NG = BPW // G
    IDXG = G * L
    chunks = []
    off = 0
    while off < IDXG:
        ln = min(128, IDXG - off)
        chunks.append((off, ln))
        off += ln

    mesh = plsc.VectorSubcoreMesh(core_axis_name="c", subcore_axis_name="s")

    @functools.partial(
        pl.kernel,
        mesh=mesh,
        out_type=jax.ShapeDtypeStruct((B * D,), jnp.float32),
        scratch_types=[
            pltpu.VMEM((IDXG,), jnp.int32),
            pltpu.VMEM((IDXG,), jnp.int32),
            pltpu.VMEM((IDXG, D2), jnp.float32),
            pltpu.VMEM((IDXG, D2), jnp.float32),
            pltpu.VMEM((G * D,), jnp.float32),
            pltpu.SemaphoreType.DMA,
            pltpu.SemaphoreType.DMA,
        ],
    )
    def sc_pool(idx_ref, emb2_ref, out_ref,
                idx0, idx1, rows0, rows1, pooled,
                sem0, sem1):
        wid = lax.axis_index("s") * _NUM_CORES + lax.axis_index("c")
        row_base = wid * BPW
        idx_base = row_base * L
        idxs = (idx0, idx1)
        rows = (rows0, rows1)
        sems = (sem0, sem1)

        def issue(g, buf):
            off = pl.multiple_of(idx_base + g * IDXG, 8)
            pltpu.sync_copy(idx_ref.at[pl.ds(off, IDXG)], idxs[buf])
            for (o, ln) in chunks:
                pltpu.async_copy(
                    emb2_ref.at[idxs[buf].at[pl.ds(o, ln)]],
                    rows[buf].at[pl.ds(o, ln)],
                    sems[buf])

        def wait(buf):
            for (o, ln) in chunks:
                pltpu.make_async_copy(
                    emb2_ref.at[idxs[buf].at[pl.ds(o, ln)]],
                    rows[buf].at[pl.ds(o, ln)],
                    sems[buf]).wait()

        def reduce_store(g, buf):
            r = rows[buf]

            def body(c, carry):
                accs = None
                for l in range(L):
                    s = c * L + l
                    cur = [r[s, pl.ds(j * 16, 16)] for j in range(D // 16)]
                    if accs is None:
                        accs = cur
                    else:
                        accs = [a + b for a, b in zip(accs, cur)]
                for j in range(D // 16):
                    pooled[pl.ds(pl.multiple_of(c * D + j * 16, 16), 16)] = (
                        accs[j])
                return carry

            lax.fori_loop(0, G, body, 0)
            out_off = pl.multiple_of((row_base + g * G) * D, 8)
            pltpu.sync_copy(pooled, out_ref.at[pl.ds(out_off, G * D)])

        issue(0, 0)

        def pair_body(p, carry):
            g0 = 2 * p
            issue(g0 + 1, 1)
            wait(0)
            reduce_store(g0, 0)

            @pl.when(g0 + 2 < NG)
            def _():
                issue(g0 + 2, 0)

            wait(1)
            reduce_store(g0 + 1, 1)
            return carry

        lax.fori_loop(0, NG // 2, pair_body, 0)

    return sc_pool


@functools.cache
def _make_sc_transpose(V, D):
    D2 = 2 * D
    K = 128
    NBF = V // K
    VTAIL = V - NBF * K
    NMAIN = (NBF // _NUM_WORKERS) * _NUM_WORKERS
    KPW = NMAIN // _NUM_WORKERS
    NEXTRA = NBF - NMAIN

    mesh = plsc.VectorSubcoreMesh(core_axis_name="c", subcore_axis_name="s")

    @functools.partial(
        pl.kernel,
        mesh=mesh,
        out_type=jax.ShapeDtypeStruct((V, D2), jnp.float32),
        scratch_types=[
            pltpu.VMEM((D, K), jnp.float32),
            pltpu.VMEM((D, K), jnp.float32),
            pltpu.VMEM((K, D2), jnp.float32),
            pltpu.VMEM((K, D2), jnp.float32),
            pltpu.SemaphoreType.DMA,
            pltpu.SemaphoreType.DMA,
            pltpu.SemaphoreType.DMA,
            pltpu.SemaphoreType.DMA,
        ],
    )
    def sc_t(embt_ref, tail_ref, out_ref,
             in0, in1, o0, o1, isem0, isem1, osem0, osem1):
        wid = lax.axis_index("s") * _NUM_CORES + lax.axis_index("c")
        ins = (in0, in1)
        outs = (o0, o1)
        isems = (isem0, isem1)
        osems = (osem0, osem1)
        jvecs = [lax.iota(jnp.int32, 16) + 16 * jj
                 for jj in range(D // 16)]

        def v0_of(k):
            return pl.multiple_of((k * _NUM_WORKERS + wid) * K, 8)

        def start_in(k, buf):
            pltpu.async_copy(
                embt_ref.at[:, pl.ds(v0_of(k), K)], ins[buf], isems[buf])

        def wait_in(k, buf):
            pltpu.make_async_copy(
                embt_ref.at[:, pl.ds(v0_of(k), K)], ins[buf],
                isems[buf]).wait()

        def start_out(k, buf):
            pltpu.async_copy(
                outs[buf], out_ref.at[pl.ds(v0_of(k), K)], osems[buf])

        def wait_out(k, buf):
            pltpu.make_async_copy(
                outs[buf], out_ref.at[pl.ds(v0_of(k), K)],
                osems[buf]).wait()

        def transpose_block(src, dst, nv):
            def tb(v, carry):
                for jj in range(D // 16):
                    vec = plsc.load_gather(
                        src, [jvecs[jj], jnp.full((16,), v, jnp.int32)])
                    dst[v, pl.ds(jj * 16, 16)] = vec
                return carry
            lax.fori_loop(0, nv, tb, 0)

        start_in(0, 0)

        def pair(p, carry):
            k0 = 2 * p
            start_in(k0 + 1, 1)
            wait_in(k0, 0)

            @pl.when(k0 >= 2)
            def _():
                wait_out(k0 - 2, 0)

            transpose_block(ins[0], outs[0], K)
            start_out(k0, 0)

            @pl.when(k0 + 2 < KPW)
            def _():
                start_in(k0 + 2, 0)

            wait_in(k0 + 1, 1)

            @pl.when(k0 >= 1)
            def _():
                wait_out(k0 - 1, 1)

            transpose_block(ins[1], outs[1], K)
            start_out(k0 + 1, 1)
            return carry

        lax.fori_loop(0, KPW // 2, pair, 0)
        wait_out(KPW - 2, 0)
        wait_out(KPW - 1, 1)

        @pl.when(wid < NEXTRA)
        def _():
            v0 = pl.multiple_of((NMAIN + wid) * K, 8)
            pltpu.sync_copy(embt_ref.at[:, pl.ds(v0, K)], ins[0])
            transpose_block(ins[0], outs[0], K)
            pltpu.sync_copy(outs[0], out_ref.at[pl.ds(v0, K)])

        @pl.when(wid == NEXTRA)
        def _():
            pltpu.sync_copy(tail_ref, ins[0])
            transpose_block(ins[0], outs[0], VTAIL)
            pltpu.sync_copy(
                outs[0].at[pl.ds(0, VTAIL)],
                out_ref.at[pl.ds(pl.multiple_of(NBF * K, 8), VTAIL)])

    return sc_t


@functools.cache
def _make_tc_transpose(V, D):
    VB = 4096

    def body(x_ref, o_ref):
        y = x_ref[...].T
        o_ref[...] = jnp.concatenate([y, jnp.zeros_like(y)], axis=1)

    return pl.pallas_call(
        body,
        grid=((V + VB - 1) // VB,),
        in_specs=[pl.BlockSpec((D, VB), lambda i: (0, i))],
        out_specs=pl.BlockSpec((VB, 2 * D), lambda i: (i, 0)),
        out_shape=jax.ShapeDtypeStruct((V, 2 * D), jnp.float32),
    )


@functools.cache
def _make_mlp(B, D, H1, H2):
    BT = 512

    def body(x_ref, w1_ref, b1_ref, w2_ref, b2_ref, w3_ref, b3_ref, o_ref):
        x = x_ref[...]
        h1 = jnp.maximum(
            jnp.dot(x, w1_ref[...], preferred_element_type=jnp.float32)
            + b1_ref[...], 0.0)
        h2 = jnp.maximum(
            jnp.dot(h1, w2_ref[...], preferred_element_type=jnp.float32)
            + b2_ref[...], 0.0)
        o_ref[...] = (
            jnp.dot(h2, w3_ref[...], preferred_element_type=jnp.float32)
            + b3_ref[...])

    return pl.pallas_call(
        body,
        grid=(B // BT,),
        in_specs=[
            pl.BlockSpec((BT, D), lambda i: (i, 0)),
            pl.BlockSpec((D, H1), lambda i: (0, 0)),
            pl.BlockSpec((1, H1), lambda i: (0, 0)),
            pl.BlockSpec((H1, H2), lambda i: (0, 0)),
            pl.BlockSpec((1, H2), lambda i: (0, 0)),
            pl.BlockSpec((H2, 128), lambda i: (0, 0)),
            pl.BlockSpec((1, 128), lambda i: (0, 0)),
        ],
        out_specs=pl.BlockSpec((BT, 128), lambda i: (i, 0)),
        out_shape=jax.ShapeDtypeStruct((B, 128), jnp.float32),
    )


def kernel(sentence, emb, W1, b1, W2, b2, W3, b3):
    B, L = sentence.shape
    V, D = emb.shape
    H1 = W1.shape[0]
    H2 = W2.shape[0]
    NOUT = W3.shape[0]

    idx = sentence.astype(jnp.int32).reshape(-1)
    emb2 = _make_tc_transpose(V, D)(emb.astype(jnp.float32).T)
    pooled = _make_sc_pool(B, L, D)(idx, emb2).reshape(B, D)

    w3t = jnp.zeros((H2, 128), jnp.float32).at[:, :NOUT].set(W3.T)
    b3p = jnp.zeros((1, 128), jnp.float32).at[0, :NOUT].set(b3)
    out = _make_mlp(B, D, H1, H2)(
        pooled, W1.T, b1.reshape(1, H1), W2.T, b2.reshape(1, H2), w3t, b3p)
    return out[:, :NOUT]


# speedup vs baseline: 1.6272x; 1.2064x over previous
import functools

import jax
import jax.numpy as jnp
from jax import lax
from jax.experimental import pallas as pl
from jax.experimental.pallas import tpu as pltpu
from jax.experimental.pallas import tpu_sc as plsc

_NUM_CORES = 2
_NUM_SUBCORES = 16
_NUM_WORKERS = _NUM_CORES * _NUM_SUBCORES


@functools.cache
def _make_sc_pool(B, L, D):
    D2 = 2 * D
    BPW = B // _NUM_WORKERS
    G = 4
    NG = BPW // G
    IDXG = G * L
    chunks = []
    off = 0
    while off < IDXG:
        ln = min(128, IDXG - off)
        chunks.append((off, ln))
        off += ln

    mesh = plsc.VectorSubcoreMesh(core_axis_name="c", subcore_axis_name="s")

    @functools.partial(
        pl.kernel,
        mesh=mesh,
        out_type=jax.ShapeDtypeStruct((B * D,), jnp.float32),
        scratch_types=[
            pltpu.VMEM((IDXG,), jnp.int32),
            pltpu.VMEM((IDXG,), jnp.int32),
            pltpu.VMEM((IDXG, D2), jnp.float32),
            pltpu.VMEM((IDXG, D2), jnp.float32),
            pltpu.VMEM((G * D,), jnp.float32),
            pltpu.SemaphoreType.DMA,
            pltpu.SemaphoreType.DMA,
        ],
    )
    def sc_pool(idx_ref, emb2_ref, out_ref,
                idx0, idx1, rows0, rows1, pooled,
                sem0, sem1):
        wid = lax.axis_index("s") * _NUM_CORES + lax.axis_index("c")
        row_base = wid * BPW
        idx_base = row_base * L
        idxs = (idx0, idx1)
        rows = (rows0, rows1)
        sems = (sem0, sem1)

        def issue(g, buf):
            off = pl.multiple_of(idx_base + g * IDXG, 8)
            pltpu.sync_copy(idx_ref.at[pl.ds(off, IDXG)], idxs[buf])
            for (o, ln) in chunks:
                pltpu.async_copy(
                    emb2_ref.at[idxs[buf].at[pl.ds(o, ln)]],
                    rows[buf].at[pl.ds(o, ln)],
                    sems[buf])

        def wait(buf):
            for (o, ln) in chunks:
                pltpu.make_async_copy(
                    emb2_ref.at[idxs[buf].at[pl.ds(o, ln)]],
                    rows[buf].at[pl.ds(o, ln)],
                    sems[buf]).wait()

        def reduce_store(g, buf):
            r = rows[buf]

            def body(c, carry):
                accs = None
                for l in range(L):
                    s = c * L + l
                    cur = [r[s, pl.ds(j * 16, 16)] for j in range(D // 16)]
                    if accs is None:
                        accs = cur
                    else:
                        accs = [a + b for a, b in zip(accs, cur)]
                for j in range(D // 16):
                    pooled[pl.ds(pl.multiple_of(c * D + j * 16, 16), 16)] = (
                        accs[j])
                return carry

            lax.fori_loop(0, G, body, 0)
            out_off = pl.multiple_of((row_base + g * G) * D, 8)
            pltpu.sync_copy(pooled, out_ref.at[pl.ds(out_off, G * D)])

        issue(0, 0)

        def pair_body(p, carry):
            g0 = 2 * p
            issue(g0 + 1, 1)
            wait(0)
            reduce_store(g0, 0)

            @pl.when(g0 + 2 < NG)
            def _():
                issue(g0 + 2, 0)

            wait(1)
            reduce_store(g0 + 1, 1)
            return carry

        lax.fori_loop(0, NG // 2, pair_body, 0)

    return sc_pool


@functools.cache
def _make_sc_transpose(V, D):
    D2 = 2 * D
    K = 128
    NBF = V // K
    VTAIL = V - NBF * K
    NMAIN = (NBF // _NUM_WORKERS) * _NUM_WORKERS
    KPW = NMAIN // _NUM_WORKERS
    NEXTRA = NBF - NMAIN

    mesh = plsc.VectorSubcoreMesh(core_axis_name="c", subcore_axis_name="s")

    @functools.partial(
        pl.kernel,
        mesh=mesh,
        out_type=jax.ShapeDtypeStruct((V, D2), jnp.float32),
        scratch_types=[
            pltpu.VMEM((D, K), jnp.float32),
            pltpu.VMEM((D, K), jnp.float32),
            pltpu.VMEM((K, D2), jnp.float32),
            pltpu.VMEM((K, D2), jnp.float32),
            pltpu.SemaphoreType.DMA,
            pltpu.SemaphoreType.DMA,
            pltpu.SemaphoreType.DMA,
            pltpu.SemaphoreType.DMA,
        ],
    )
    def sc_t(embt_ref, tail_ref, out_ref,
             in0, in1, o0, o1, isem0, isem1, osem0, osem1):
        wid = lax.axis_index("s") * _NUM_CORES + lax.axis_index("c")
        ins = (in0, in1)
        outs = (o0, o1)
        isems = (isem0, isem1)
        osems = (osem0, osem1)
        jvecs = [lax.iota(jnp.int32, 16) + 16 * jj
                 for jj in range(D // 16)]

        def v0_of(k):
            return pl.multiple_of((k * _NUM_WORKERS + wid) * K, 8)

        def start_in(k, buf):
            pltpu.async_copy(
                embt_ref.at[:, pl.ds(v0_of(k), K)], ins[buf], isems[buf])

        def wait_in(k, buf):
            pltpu.make_async_copy(
                embt_ref.at[:, pl.ds(v0_of(k), K)], ins[buf],
                isems[buf]).wait()

        def start_out(k, buf):
            pltpu.async_copy(
                outs[buf], out_ref.at[pl.ds(v0_of(k), K)], osems[buf])

        def wait_out(k, buf):
            pltpu.make_async_copy(
                outs[buf], out_ref.at[pl.ds(v0_of(k), K)],
                osems[buf]).wait()

        def transpose_block(src, dst, nv):
            def tb(v, carry):
                for jj in range(D // 16):
                    vec = plsc.load_gather(
                        src, [jvecs[jj], jnp.full((16,), v, jnp.int32)])
                    dst[v, pl.ds(jj * 16, 16)] = vec
                return carry
            lax.fori_loop(0, nv, tb, 0)

        start_in(0, 0)

        def pair(p, carry):
            k0 = 2 * p
            start_in(k0 + 1, 1)
            wait_in(k0, 0)

            @pl.when(k0 >= 2)
            def _():
                wait_out(k0 - 2, 0)

            transpose_block(ins[0], outs[0], K)
            start_out(k0, 0)

            @pl.when(k0 + 2 < KPW)
            def _():
                start_in(k0 + 2, 0)

            wait_in(k0 + 1, 1)

            @pl.when(k0 >= 1)
            def _():
                wait_out(k0 - 1, 1)

            transpose_block(ins[1], outs[1], K)
            start_out(k0 + 1, 1)
            return carry

        lax.fori_loop(0, KPW // 2, pair, 0)
        wait_out(KPW - 2, 0)
        wait_out(KPW - 1, 1)

        @pl.when(wid < NEXTRA)
        def _():
            v0 = pl.multiple_of((NMAIN + wid) * K, 8)
            pltpu.sync_copy(embt_ref.at[:, pl.ds(v0, K)], ins[0])
            transpose_block(ins[0], outs[0], K)
            pltpu.sync_copy(outs[0], out_ref.at[pl.ds(v0, K)])

        @pl.when(wid == NEXTRA)
        def _():
            pltpu.sync_copy(tail_ref, ins[0])
            transpose_block(ins[0], outs[0], VTAIL)
            pltpu.sync_copy(
                outs[0].at[pl.ds(0, VTAIL)],
                out_ref.at[pl.ds(pl.multiple_of(NBF * K, 8), VTAIL)])

    return sc_t


@functools.cache
def _make_tc_transpose(V, D):
    VB = 8192

    def body(x_ref, o_ref):
        y = x_ref[...].T
        o_ref[...] = jnp.concatenate([y, jnp.zeros_like(y)], axis=1)

    return pl.pallas_call(
        body,
        grid=((V + VB - 1) // VB,),
        in_specs=[pl.BlockSpec((D, VB), lambda i: (0, i))],
        out_specs=pl.BlockSpec((VB, 2 * D), lambda i: (i, 0)),
        out_shape=jax.ShapeDtypeStruct((V, 2 * D), jnp.float32),
    )


@functools.cache
def _make_mlp(B, D, H1, H2):
    BT = 512

    def body(x_ref, w1_ref, b1_ref, w2_ref, b2_ref, w3_ref, b3_ref, o_ref):
        x = x_ref[...]
        h1 = jnp.maximum(
            jnp.dot(x, w1_ref[...], preferred_element_type=jnp.float32)
            + b1_ref[...], 0.0)
        h2 = jnp.maximum(
            jnp.dot(h1, w2_ref[...], preferred_element_type=jnp.float32)
            + b2_ref[...], 0.0)
        o_ref[...] = (
            jnp.dot(h2, w3_ref[...], preferred_element_type=jnp.float32)
            + b3_ref[...])

    return pl.pallas_call(
        body,
        grid=(B // BT,),
        in_specs=[
            pl.BlockSpec((BT, D), lambda i: (i, 0)),
            pl.BlockSpec((D, H1), lambda i: (0, 0)),
            pl.BlockSpec((1, H1), lambda i: (0, 0)),
            pl.BlockSpec((H1, H2), lambda i: (0, 0)),
            pl.BlockSpec((1, H2), lambda i: (0, 0)),
            pl.BlockSpec((H2, 128), lambda i: (0, 0)),
            pl.BlockSpec((1, 128), lambda i: (0, 0)),
        ],
        out_specs=pl.BlockSpec((BT, 128), lambda i: (i, 0)),
        out_shape=jax.ShapeDtypeStruct((B, 128), jnp.float32),
    )


def kernel(sentence, emb, W1, b1, W2, b2, W3, b3):
    B, L = sentence.shape
    V, D = emb.shape
    H1 = W1.shape[0]
    H2 = W2.shape[0]
    NOUT = W3.shape[0]

    idx = sentence.astype(jnp.int32).reshape(-1)
    emb2 = _make_tc_transpose(V, D)(emb.astype(jnp.float32).T)
    pooled = _make_sc_pool(B, L, D)(idx, emb2).reshape(B, D)

    w3t = jnp.zeros((H2, 128), jnp.float32).at[:, :NOUT].set(W3.T)
    b3p = jnp.zeros((1, 128), jnp.float32).at[0, :NOUT].set(b3)
    out = _make_mlp(B, D, H1, H2)(
        pooled, W1.T, b1.reshape(1, H1), W2.T, b2.reshape(1, H2), w3t, b3p)
    return out[:, :NOUT]


# speedup vs baseline: 1.7242x; 1.0596x over previous
import functools

import jax
import jax.numpy as jnp
from jax import lax
from jax.experimental import pallas as pl
from jax.experimental.pallas import tpu as pltpu
from jax.experimental.pallas import tpu_sc as plsc

_NUM_CORES = 2
_NUM_SUBCORES = 16
_NUM_WORKERS = _NUM_CORES * _NUM_SUBCORES


@functools.cache
def _make_sc_pool(B, L, D):
    D2 = 2 * D
    BPW = B // _NUM_WORKERS
    G = 4
    NG = BPW // G
    IDXG = G * L
    chunks = []
    off = 0
    while off < IDXG:
        ln = min(128, IDXG - off)
        chunks.append((off, ln))
        off += ln

    mesh = plsc.VectorSubcoreMesh(core_axis_name="c", subcore_axis_name="s")

    @functools.partial(
        pl.kernel,
        mesh=mesh,
        out_type=jax.ShapeDtypeStruct((B * D,), jnp.float32),
        scratch_types=[
            pltpu.VMEM((IDXG,), jnp.int32),
            pltpu.VMEM((IDXG,), jnp.int32),
            pltpu.VMEM((IDXG, D2), jnp.float32),
            pltpu.VMEM((IDXG, D2), jnp.float32),
            pltpu.VMEM((G * D,), jnp.float32),
            pltpu.SemaphoreType.DMA,
            pltpu.SemaphoreType.DMA,
        ],
    )
    def sc_pool(idx_ref, emb2_ref, out_ref,
                idx0, idx1, rows0, rows1, pooled,
                sem0, sem1):
        wid = lax.axis_index("s") * _NUM_CORES + lax.axis_index("c")
        row_base = wid * BPW
        idx_base = row_base * L
        idxs = (idx0, idx1)
        rows = (rows0, rows1)
        sems = (sem0, sem1)

        def issue(g, buf):
            off = pl.multiple_of(idx_base + g * IDXG, 8)
            pltpu.sync_copy(idx_ref.at[pl.ds(off, IDXG)], idxs[buf])
            for (o, ln) in chunks:
                pltpu.async_copy(
                    emb2_ref.at[idxs[buf].at[pl.ds(o, ln)]],
                    rows[buf].at[pl.ds(o, ln)],
                    sems[buf])

        def wait(buf):
            for (o, ln) in chunks:
                pltpu.make_async_copy(
                    emb2_ref.at[idxs[buf].at[pl.ds(o, ln)]],
                    rows[buf].at[pl.ds(o, ln)],
                    sems[buf]).wait()

        def reduce_store(g, buf):
            r = rows[buf]

            def body(c, carry):
                accs = None
                for l in range(L):
                    s = c * L + l
                    cur = [r[s, pl.ds(j * 16, 16)] for j in range(D // 16)]
                    if accs is None:
                        accs = cur
                    else:
                        accs = [a + b for a, b in zip(accs, cur)]
                for j in range(D // 16):
                    pooled[pl.ds(pl.multiple_of(c * D + j * 16, 16), 16)] = (
                        accs[j])
                return carry

            lax.fori_loop(0, G, body, 0)
            out_off = pl.multiple_of((row_base + g * G) * D, 8)
            pltpu.sync_copy(pooled, out_ref.at[pl.ds(out_off, G * D)])

        issue(0, 0)

        def pair_body(p, carry):
            g0 = 2 * p
            issue(g0 + 1, 1)
            wait(0)
            reduce_store(g0, 0)

            @pl.when(g0 + 2 < NG)
            def _():
                issue(g0 + 2, 0)

            wait(1)
            reduce_store(g0 + 1, 1)
            return carry

        lax.fori_loop(0, NG // 2, pair_body, 0)

    return sc_pool


@functools.cache
def _make_sc_transpose(V, D):
    D2 = 2 * D
    K = 128
    NBF = V // K
    VTAIL = V - NBF * K
    NMAIN = (NBF // _NUM_WORKERS) * _NUM_WORKERS
    KPW = NMAIN // _NUM_WORKERS
    NEXTRA = NBF - NMAIN

    mesh = plsc.VectorSubcoreMesh(core_axis_name="c", subcore_axis_name="s")

    @functools.partial(
        pl.kernel,
        mesh=mesh,
        out_type=jax.ShapeDtypeStruct((V, D2), jnp.float32),
        scratch_types=[
            pltpu.VMEM((D, K), jnp.float32),
            pltpu.VMEM((D, K), jnp.float32),
            pltpu.VMEM((K, D2), jnp.float32),
            pltpu.VMEM((K, D2), jnp.float32),
            pltpu.SemaphoreType.DMA,
            pltpu.SemaphoreType.DMA,
            pltpu.SemaphoreType.DMA,
            pltpu.SemaphoreType.DMA,
        ],
    )
    def sc_t(embt_ref, tail_ref, out_ref,
             in0, in1, o0, o1, isem0, isem1, osem0, osem1):
        wid = lax.axis_index("s") * _NUM_CORES + lax.axis_index("c")
        ins = (in0, in1)
        outs = (o0, o1)
        isems = (isem0, isem1)
        osems = (osem0, osem1)
        jvecs = [lax.iota(jnp.int32, 16) + 16 * jj
                 for jj in range(D // 16)]

        def v0_of(k):
            return pl.multiple_of((k * _NUM_WORKERS + wid) * K, 8)

        def start_in(k, buf):
            pltpu.async_copy(
                embt_ref.at[:, pl.ds(v0_of(k), K)], ins[buf], isems[buf])

        def wait_in(k, buf):
            pltpu.make_async_copy(
                embt_ref.at[:, pl.ds(v0_of(k), K)], ins[buf],
                isems[buf]).wait()

        def start_out(k, buf):
            pltpu.async_copy(
                outs[buf], out_ref.at[pl.ds(v0_of(k), K)], osems[buf])

        def wait_out(k, buf):
            pltpu.make_async_copy(
                outs[buf], out_ref.at[pl.ds(v0_of(k), K)],
                osems[buf]).wait()

        def transpose_block(src, dst, nv):
            def tb(v, carry):
                for jj in range(D // 16):
                    vec = plsc.load_gather(
                        src, [jvecs[jj], jnp.full((16,), v, jnp.int32)])
                    dst[v, pl.ds(jj * 16, 16)] = vec
                return carry
            lax.fori_loop(0, nv, tb, 0)

        start_in(0, 0)

        def pair(p, carry):
            k0 = 2 * p
            start_in(k0 + 1, 1)
            wait_in(k0, 0)

            @pl.when(k0 >= 2)
            def _():
                wait_out(k0 - 2, 0)

            transpose_block(ins[0], outs[0], K)
            start_out(k0, 0)

            @pl.when(k0 + 2 < KPW)
            def _():
                start_in(k0 + 2, 0)

            wait_in(k0 + 1, 1)

            @pl.when(k0 >= 1)
            def _():
                wait_out(k0 - 1, 1)

            transpose_block(ins[1], outs[1], K)
            start_out(k0 + 1, 1)
            return carry

        lax.fori_loop(0, KPW // 2, pair, 0)
        wait_out(KPW - 2, 0)
        wait_out(KPW - 1, 1)

        @pl.when(wid < NEXTRA)
        def _():
            v0 = pl.multiple_of((NMAIN + wid) * K, 8)
            pltpu.sync_copy(embt_ref.at[:, pl.ds(v0, K)], ins[0])
            transpose_block(ins[0], outs[0], K)
            pltpu.sync_copy(outs[0], out_ref.at[pl.ds(v0, K)])

        @pl.when(wid == NEXTRA)
        def _():
            pltpu.sync_copy(tail_ref, ins[0])
            transpose_block(ins[0], outs[0], VTAIL)
            pltpu.sync_copy(
                outs[0].at[pl.ds(0, VTAIL)],
                out_ref.at[pl.ds(pl.multiple_of(NBF * K, 8), VTAIL)])

    return sc_t


@functools.cache
def _make_tc_transpose(V, D):
    VB = 16384

    def body(x_ref, o_ref):
        y = x_ref[...].T
        o_ref[...] = jnp.concatenate([y, jnp.zeros_like(y)], axis=1)

    return pl.pallas_call(
        body,
        grid=((V + VB - 1) // VB,),
        in_specs=[pl.BlockSpec((D, VB), lambda i: (0, i))],
        out_specs=pl.BlockSpec((VB, 2 * D), lambda i: (i, 0)),
        out_shape=jax.ShapeDtypeStruct((V, 2 * D), jnp.float32),
    )


@functools.cache
def _make_mlp(B, D, H1, H2):
    BT = 512

    def body(x_ref, w1_ref, b1_ref, w2_ref, b2_ref, w3_ref, b3_ref, o_ref):
        x = x_ref[...]
        h1 = jnp.maximum(
            jnp.dot(x, w1_ref[...], preferred_element_type=jnp.float32)
            + b1_ref[...], 0.0)
        h2 = jnp.maximum(
            jnp.dot(h1, w2_ref[...], preferred_element_type=jnp.float32)
            + b2_ref[...], 0.0)
        o_ref[...] = (
            jnp.dot(h2, w3_ref[...], preferred_element_type=jnp.float32)
            + b3_ref[...])

    return pl.pallas_call(
        body,
        grid=(B // BT,),
        in_specs=[
            pl.BlockSpec((BT, D), lambda i: (i, 0)),
            pl.BlockSpec((D, H1), lambda i: (0, 0)),
            pl.BlockSpec((1, H1), lambda i: (0, 0)),
            pl.BlockSpec((H1, H2), lambda i: (0, 0)),
            pl.BlockSpec((1, H2), lambda i: (0, 0)),
            pl.BlockSpec((H2, 128), lambda i: (0, 0)),
            pl.BlockSpec((1, 128), lambda i: (0, 0)),
        ],
        out_specs=pl.BlockSpec((BT, 128), lambda i: (i, 0)),
        out_shape=jax.ShapeDtypeStruct((B, 128), jnp.float32),
    )


def kernel(sentence, emb, W1, b1, W2, b2, W3, b3):
    B, L = sentence.shape
    V, D = emb.shape
    H1 = W1.shape[0]
    H2 = W2.shape[0]
    NOUT = W3.shape[0]

    idx = sentence.astype(jnp.int32).reshape(-1)
    emb2 = _make_tc_transpose(V, D)(emb.astype(jnp.float32).T)
    pooled = _make_sc_pool(B, L, D)(idx, emb2).reshape(B, D)

    w3t = jnp.zeros((H2, 128), jnp.float32).at[:, :NOUT].set(W3.T)
    b3p = jnp.zeros((1, 128), jnp.float32).at[0, :NOUT].set(b3)
    out = _make_mlp(B, D, H1, H2)(
        pooled, W1.T, b1.reshape(1, H1), W2.T, b2.reshape(1, H2), w3t, b3p)
    return out[:, :NOUT]


# speedup vs baseline: 2.2864x; 1.3261x over previous
import functools

import jax
import jax.numpy as jnp
from jax import lax
from jax.experimental import pallas as pl
from jax.experimental.pallas import tpu as pltpu
from jax.experimental.pallas import tpu_sc as plsc

_NUM_CORES = 2
_NUM_SUBCORES = 16
_NUM_WORKERS = _NUM_CORES * _NUM_SUBCORES


@functools.cache
def _make_sc_pool(B, L, D):
    D2 = 2 * D
    BPW = B // _NUM_WORKERS
    G = 4
    NG = BPW // G
    IDXG = G * L
    chunks = []
    off = 0
    while off < IDXG:
        ln = min(128, IDXG - off)
        chunks.append((off, ln))
        off += ln

    mesh = plsc.VectorSubcoreMesh(core_axis_name="c", subcore_axis_name="s")

    @functools.partial(
        pl.kernel,
        mesh=mesh,
        out_type=jax.ShapeDtypeStruct((B * D,), jnp.float32),
        scratch_types=[
            pltpu.VMEM((IDXG,), jnp.int32),
            pltpu.VMEM((IDXG,), jnp.int32),
            pltpu.VMEM((IDXG, D2), jnp.float32),
            pltpu.VMEM((IDXG, D2), jnp.float32),
            pltpu.VMEM((G * D,), jnp.float32),
            pltpu.SemaphoreType.DMA,
            pltpu.SemaphoreType.DMA,
        ],
    )
    def sc_pool(idx_ref, emb2_ref, out_ref,
                idx0, idx1, rows0, rows1, pooled,
                sem0, sem1):
        wid = lax.axis_index("s") * _NUM_CORES + lax.axis_index("c")
        row_base = wid * BPW
        idx_base = row_base * L
        idxs = (idx0, idx1)
        rows = (rows0, rows1)
        sems = (sem0, sem1)

        def issue(g, buf):
            off = pl.multiple_of(idx_base + g * IDXG, 8)
            pltpu.sync_copy(idx_ref.at[pl.ds(off, IDXG)], idxs[buf])
            for (o, ln) in chunks:
                pltpu.async_copy(
                    emb2_ref.at[idxs[buf].at[pl.ds(o, ln)]],
                    rows[buf].at[pl.ds(o, ln)],
                    sems[buf])

        def wait(buf):
            for (o, ln) in chunks:
                pltpu.make_async_copy(
                    emb2_ref.at[idxs[buf].at[pl.ds(o, ln)]],
                    rows[buf].at[pl.ds(o, ln)],
                    sems[buf]).wait()

        def reduce_store(g, buf):
            r = rows[buf]

            def body(c, carry):
                accs = None
                for l in range(L):
                    s = c * L + l
                    cur = [r[s, pl.ds(j * 16, 16)] for j in range(D // 16)]
                    if accs is None:
                        accs = cur
                    else:
                        accs = [a + b for a, b in zip(accs, cur)]
                for j in range(D // 16):
                    pooled[pl.ds(pl.multiple_of(c * D + j * 16, 16), 16)] = (
                        accs[j])
                return carry

            lax.fori_loop(0, G, body, 0)
            out_off = pl.multiple_of((row_base + g * G) * D, 8)
            pltpu.sync_copy(pooled, out_ref.at[pl.ds(out_off, G * D)])

        issue(0, 0)

        def pair_body(p, carry):
            g0 = 2 * p
            issue(g0 + 1, 1)
            wait(0)
            reduce_store(g0, 0)

            @pl.when(g0 + 2 < NG)
            def _():
                issue(g0 + 2, 0)

            wait(1)
            reduce_store(g0 + 1, 1)
            return carry

        lax.fori_loop(0, NG // 2, pair_body, 0)

    return sc_pool


@functools.cache
def _make_sc_transpose(V, D):
    D2 = 2 * D
    K = 128
    NBF = V // K
    VTAIL = V - NBF * K
    NMAIN = (NBF // _NUM_WORKERS) * _NUM_WORKERS
    KPW = NMAIN // _NUM_WORKERS
    NEXTRA = NBF - NMAIN

    mesh = plsc.VectorSubcoreMesh(core_axis_name="c", subcore_axis_name="s")

    @functools.partial(
        pl.kernel,
        mesh=mesh,
        out_type=jax.ShapeDtypeStruct((V, D2), jnp.float32),
        scratch_types=[
            pltpu.VMEM((D, K), jnp.float32),
            pltpu.VMEM((D, K), jnp.float32),
            pltpu.VMEM((K, D2), jnp.float32),
            pltpu.VMEM((K, D2), jnp.float32),
            pltpu.SemaphoreType.DMA,
            pltpu.SemaphoreType.DMA,
            pltpu.SemaphoreType.DMA,
            pltpu.SemaphoreType.DMA,
        ],
    )
    def sc_t(embt_ref, tail_ref, out_ref,
             in0, in1, o0, o1, isem0, isem1, osem0, osem1):
        wid = lax.axis_index("s") * _NUM_CORES + lax.axis_index("c")
        ins = (in0, in1)
        outs = (o0, o1)
        isems = (isem0, isem1)
        osems = (osem0, osem1)
        jvecs = [lax.iota(jnp.int32, 16) + 16 * jj
                 for jj in range(D // 16)]

        def v0_of(k):
            return pl.multiple_of((k * _NUM_WORKERS + wid) * K, 8)

        def start_in(k, buf):
            pltpu.async_copy(
                embt_ref.at[:, pl.ds(v0_of(k), K)], ins[buf], isems[buf])

        def wait_in(k, buf):
            pltpu.make_async_copy(
                embt_ref.at[:, pl.ds(v0_of(k), K)], ins[buf],
                isems[buf]).wait()

        def start_out(k, buf):
            pltpu.async_copy(
                outs[buf], out_ref.at[pl.ds(v0_of(k), K)], osems[buf])

        def wait_out(k, buf):
            pltpu.make_async_copy(
                outs[buf], out_ref.at[pl.ds(v0_of(k), K)],
                osems[buf]).wait()

        def transpose_block(src, dst, nv):
            def tb(v, carry):
                for jj in range(D // 16):
                    vec = plsc.load_gather(
                        src, [jvecs[jj], jnp.full((16,), v, jnp.int32)])
                    dst[v, pl.ds(jj * 16, 16)] = vec
                return carry
            lax.fori_loop(0, nv, tb, 0)

        start_in(0, 0)

        def pair(p, carry):
            k0 = 2 * p
            start_in(k0 + 1, 1)
            wait_in(k0, 0)

            @pl.when(k0 >= 2)
            def _():
                wait_out(k0 - 2, 0)

            transpose_block(ins[0], outs[0], K)
            start_out(k0, 0)

            @pl.when(k0 + 2 < KPW)
            def _():
                start_in(k0 + 2, 0)

            wait_in(k0 + 1, 1)

            @pl.when(k0 >= 1)
            def _():
                wait_out(k0 - 1, 1)

            transpose_block(ins[1], outs[1], K)
            start_out(k0 + 1, 1)
            return carry

        lax.fori_loop(0, KPW // 2, pair, 0)
        wait_out(KPW - 2, 0)
        wait_out(KPW - 1, 1)

        @pl.when(wid < NEXTRA)
        def _():
            v0 = pl.multiple_of((NMAIN + wid) * K, 8)
            pltpu.sync_copy(embt_ref.at[:, pl.ds(v0, K)], ins[0])
            transpose_block(ins[0], outs[0], K)
            pltpu.sync_copy(outs[0], out_ref.at[pl.ds(v0, K)])

        @pl.when(wid == NEXTRA)
        def _():
            pltpu.sync_copy(tail_ref, ins[0])
            transpose_block(ins[0], outs[0], VTAIL)
            pltpu.sync_copy(
                outs[0].at[pl.ds(0, VTAIL)],
                out_ref.at[pl.ds(pl.multiple_of(NBF * K, 8), VTAIL)])

    return sc_t


@functools.cache
def _make_tc_transpose(V, D):
    VB = 32768

    def body(x_ref, o_ref):
        y = x_ref[...].T
        o_ref[...] = jnp.concatenate([y, jnp.zeros_like(y)], axis=1)

    return pl.pallas_call(
        body,
        grid=((V + VB - 1) // VB,),
        in_specs=[pl.BlockSpec((D, VB), lambda i: (0, i))],
        out_specs=pl.BlockSpec((VB, 2 * D), lambda i: (i, 0)),
        out_shape=jax.ShapeDtypeStruct((V, 2 * D), jnp.float32),
    )


@functools.cache
def _make_mlp(B, D, H1, H2):
    BT = 512

    def body(x_ref, w1_ref, b1_ref, w2_ref, b2_ref, w3_ref, b3_ref, o_ref):
        x = x_ref[...]
        h1 = jnp.maximum(
            jnp.dot(x, w1_ref[...], preferred_element_type=jnp.float32)
            + b1_ref[...], 0.0)
        h2 = jnp.maximum(
            jnp.dot(h1, w2_ref[...], preferred_element_type=jnp.float32)
            + b2_ref[...], 0.0)
        o_ref[...] = (
            jnp.dot(h2, w3_ref[...], preferred_element_type=jnp.float32)
            + b3_ref[...])

    return pl.pallas_call(
        body,
        grid=(B // BT,),
        in_specs=[
            pl.BlockSpec((BT, D), lambda i: (i, 0)),
            pl.BlockSpec((D, H1), lambda i: (0, 0)),
            pl.BlockSpec((1, H1), lambda i: (0, 0)),
            pl.BlockSpec((H1, H2), lambda i: (0, 0)),
            pl.BlockSpec((1, H2), lambda i: (0, 0)),
            pl.BlockSpec((H2, 128), lambda i: (0, 0)),
            pl.BlockSpec((1, 128), lambda i: (0, 0)),
        ],
        out_specs=pl.BlockSpec((BT, 128), lambda i: (i, 0)),
        out_shape=jax.ShapeDtypeStruct((B, 128), jnp.float32),
    )


def kernel(sentence, emb, W1, b1, W2, b2, W3, b3):
    B, L = sentence.shape
    V, D = emb.shape
    H1 = W1.shape[0]
    H2 = W2.shape[0]
    NOUT = W3.shape[0]

    idx = sentence.astype(jnp.int32).reshape(-1)
    emb2 = _make_tc_transpose(V, D)(emb.astype(jnp.float32).T)
    pooled = _make_sc_pool(B, L, D)(idx, emb2).reshape(B, D)

    w3t = jnp.zeros((H2, 128), jnp.float32).at[:, :NOUT].set(W3.T)
    b3p = jnp.zeros((1, 128), jnp.float32).at[0, :NOUT].set(b3)
    out = _make_mlp(B, D, H1, H2)(
        pooled, W1.T, b1.reshape(1, H1), W2.T, b2.reshape(1, H2), w3t, b3p)
    return out[:, :NOUT]
